# baseline (speedup 1.0000x reference)
import functools

import jax
import jax.numpy as jnp
from jax import lax
from jax.experimental import pallas as pl
from jax.experimental.pallas import tpu as pltpu
from jax.experimental.pallas import tpu_sc as plsc

_NUM_EXPERTS = 64
_BLOCK_M = 2048
_N_TOKENS = 16384
_N_WORKERS = 32
_TOK_PER_W = _N_TOKENS // _N_WORKERS
_GROUPS = _TOK_PER_W // 16


def _matmul_body(x_ref, w_ref, o_ref):
    o_ref[...] = jax.lax.dot_general(
        x_ref[...], w_ref[...], (((1,), (1,)), ((), ())),
        preferred_element_type=jnp.float32,
    )


def _logits_tc(x, W_g):
    n_tokens, dim = x.shape
    return pl.pallas_call(
        _matmul_body,
        grid=(n_tokens // _BLOCK_M,),
        in_specs=[
            pl.BlockSpec((_BLOCK_M, dim), lambda i: (i, 0)),
            pl.BlockSpec((_NUM_EXPERTS, dim), lambda i: (0, 0)),
        ],
        out_specs=pl.BlockSpec((_BLOCK_M, _NUM_EXPERTS), lambda i: (i, 0)),
        out_shape=jax.ShapeDtypeStruct((n_tokens, _NUM_EXPERTS), jnp.float32),
        compiler_params=pltpu.CompilerParams(
            dimension_semantics=("arbitrary",),
            vmem_limit_bytes=64 * 1024 * 1024,
        ),
    )(x, W_g)


def _sc_topk_body(logits_hbm, out_hbm, logits_v, out_v):
    wid = lax.axis_index("s") * 2 + lax.axis_index("c")
    base = wid * _TOK_PER_W
    pltpu.sync_copy(logits_hbm.at[pl.ds(base, _TOK_PER_W)], logits_v)
    lane = lax.broadcasted_iota(jnp.int32, (16,), 0)
    zeros = jnp.zeros((16,), jnp.int32)
    neg = jnp.full((16,), -jnp.inf, jnp.float32)

    def group_body(g, carry):
        row = g * 16 + lane

        def expert_body(e, vv):
            v1, v2 = vv
            x = plsc.load_gather(logits_v, [row, zeros + e])
            return jnp.maximum(v1, x), jnp.maximum(v2, jnp.minimum(v1, x))

        v1, v2 = lax.fori_loop(0, _NUM_EXPERTS, expert_body, (neg, neg))
        e2 = jnp.exp(v2 - v1)
        denom = 1.0 + e2
        plsc.store_scatter(out_v, [row, zeros], 1.0 / denom)
        plsc.store_scatter(out_v, [row, zeros + 1], e2 / denom)
        return carry

    lax.fori_loop(0, _GROUPS, group_body, 0)
    pltpu.sync_copy(out_v, out_hbm.at[pl.ds(base, _TOK_PER_W)])


_sc_topk = pl.kernel(
    _sc_topk_body,
    out_type=jax.ShapeDtypeStruct((_N_TOKENS, 2), jnp.float32),
    mesh=plsc.VectorSubcoreMesh(core_axis_name="c", subcore_axis_name="s"),
    compiler_params=pltpu.CompilerParams(needs_layout_passes=False),
    scratch_types=[
        pltpu.VMEM((_TOK_PER_W, _NUM_EXPERTS), jnp.float32),
        pltpu.VMEM((_TOK_PER_W, 2), jnp.float32),
    ],
)


@jax.jit
def kernel(x, W_g):
    return _sc_topk(_logits_tc(x, W_g))


# speedup vs baseline: 1.4198x; 1.4198x over previous
import functools

import jax
import jax.numpy as jnp
from jax.experimental import pallas as pl
from jax.experimental.pallas import tpu as pltpu

_NUM_EXPERTS = 64
_BLOCK_M = 2048


def _gating_body(x_ref, w_ref, p1_ref, p2_ref):
    x = x_ref[...]
    w = w_ref[...]
    logits = jax.lax.dot_general(
        x, w, (((1,), (1,)), ((), ())), preferred_element_type=jnp.float32
    )
    v1 = jnp.max(logits, axis=-1, keepdims=True)
    iota = jax.lax.broadcasted_iota(jnp.int32, logits.shape, 1)
    idx1 = jnp.min(
        jnp.where(logits == v1, iota, _NUM_EXPERTS), axis=-1, keepdims=True
    )
    v2 = jnp.max(jnp.where(iota == idx1, -jnp.inf, logits), axis=-1, keepdims=True)
    e2 = jnp.exp(v2 - v1)
    denom = 1.0 + e2
    p1_ref[...] = (1.0 / denom)[:, 0]
    p2_ref[...] = (e2 / denom)[:, 0]


@functools.partial(jax.jit, static_argnames=("interpret",))
def kernel(x, W_g, interpret=False):
    n_tokens, dim = x.shape
    p1, p2 = pl.pallas_call(
        _gating_body,
        grid=(n_tokens // _BLOCK_M,),
        in_specs=[
            pl.BlockSpec((_BLOCK_M, dim), lambda i: (i, 0)),
            pl.BlockSpec((_NUM_EXPERTS, dim), lambda i: (0, 0)),
        ],
        out_specs=[
            pl.BlockSpec((_BLOCK_M,), lambda i: (i,)),
            pl.BlockSpec((_BLOCK_M,), lambda i: (i,)),
        ],
        out_shape=[
            jax.ShapeDtypeStruct((n_tokens,), jnp.float32),
            jax.ShapeDtypeStruct((n_tokens,), jnp.float32),
        ],
        compiler_params=pltpu.CompilerParams(
            dimension_semantics=("arbitrary",),
            vmem_limit_bytes=64 * 1024 * 1024,
        ),
        interpret=interpret,
    )(x, W_g)
    return jnp.stack([p1, p2], axis=-1)
